# Initial kernel scaffold; baseline (speedup 1.0000x reference)
import jax
import jax.numpy as jnp
from jax.experimental import pallas as pl


def kernel(x, W):
    raise NotImplementedError("write your pallas kernel here")


# speedup vs baseline: 1.4938x; 1.4938x over previous
import functools

import jax
import jax.numpy as jnp
from jax.experimental import pallas as pl
from jax.experimental.pallas import tpu as pltpu

D_MODEL = 2048
N_EXP = 64
N_TOK = 16384
BLK = 512


def _gate_kernel(x_ref, w_ref, idx_ref, cw_ref, psum_ref, ent_ref):
    i = pl.program_id(0)
    xb = x_ref[...]
    wt = w_ref[...]
    logits = jnp.dot(xb, wt, preferred_element_type=jnp.float32)

    iota = jax.lax.broadcasted_iota(jnp.int32, logits.shape, 1)
    big = jnp.int32(N_EXP)

    m1 = jnp.max(logits, axis=-1, keepdims=True)
    i1 = jnp.min(jnp.where(logits == m1, iota, big), axis=-1, keepdims=True)
    l2 = jnp.where(iota == i1, -jnp.inf, logits)
    m2 = jnp.max(l2, axis=-1, keepdims=True)
    i2 = jnp.min(jnp.where(l2 == m2, iota, big), axis=-1, keepdims=True)

    e = jnp.exp(m2 - m1)
    denom = 1.0 + e
    c1 = 1.0 / denom
    c2 = e / denom

    idx_ref[...] = jnp.concatenate([i1, i2], axis=1)
    cw_ref[...] = jnp.concatenate([c1, c2], axis=1)

    z = logits - m1
    ez = jnp.exp(z)
    s = jnp.sum(ez, axis=-1, keepdims=True)
    p = ez / s
    psum = jnp.sum(p, axis=0)[None, :]
    ent = jnp.sum(jnp.log(s) - jnp.sum(p * z, axis=-1, keepdims=True))

    @pl.when(i == 0)
    def _init():
        psum_ref[...] = psum
        ent_ref[...] = jnp.full((1, 1), ent, jnp.float32)

    @pl.when(i != 0)
    def _acc():
        psum_ref[...] += psum
        ent_ref[...] += jnp.full((1, 1), ent, jnp.float32)


@jax.jit
def kernel(x, W):
    n_tok = x.shape[0]
    wt = W.T
    grid = (n_tok // BLK,)
    idx, cw, psum, ent = pl.pallas_call(
        _gate_kernel,
        grid=grid,
        in_specs=[
            pl.BlockSpec((BLK, D_MODEL), lambda i: (i, 0)),
            pl.BlockSpec((D_MODEL, N_EXP), lambda i: (0, 0)),
        ],
        out_specs=[
            pl.BlockSpec((BLK, 2), lambda i: (i, 0)),
            pl.BlockSpec((BLK, 2), lambda i: (i, 0)),
            pl.BlockSpec((1, N_EXP), lambda i: (0, 0)),
            pl.BlockSpec((1, 1), lambda i: (0, 0)),
        ],
        out_shape=[
            jax.ShapeDtypeStruct((n_tok, 2), jnp.int32),
            jax.ShapeDtypeStruct((n_tok, 2), jnp.float32),
            jax.ShapeDtypeStruct((1, N_EXP), jnp.float32),
            jax.ShapeDtypeStruct((1, 1), jnp.float32),
        ],
        compiler_params=pltpu.CompilerParams(
            dimension_semantics=("arbitrary",),
        ),
    )(x, wt)
    inv_n = jnp.float32(1.0 / n_tok)
    return idx, cw, psum[0] * inv_n, ent[0, 0] * inv_n


# speedup vs baseline: 1.7119x; 1.1460x over previous
import functools

import jax
import jax.numpy as jnp
from jax.experimental import pallas as pl
from jax.experimental.pallas import tpu as pltpu

D_MODEL = 2048
N_EXP = 64
N_TOK = 16384
BLK = 1024


def _gate_kernel(x_ref, w_ref, idx_ref, cw_ref, psum_ref, ent_ref):
    i = pl.program_id(0)
    xb = x_ref[...]
    wt = w_ref[...]
    logits = jnp.dot(xb, wt, preferred_element_type=jnp.float32)

    iota = jax.lax.broadcasted_iota(jnp.int32, logits.shape, 1)
    big = jnp.int32(N_EXP)

    m1 = jnp.max(logits, axis=-1, keepdims=True)
    i1 = jnp.min(jnp.where(logits == m1, iota, big), axis=-1, keepdims=True)
    l2 = jnp.where(iota == i1, -jnp.inf, logits)
    m2 = jnp.max(l2, axis=-1, keepdims=True)
    i2 = jnp.min(jnp.where(l2 == m2, iota, big), axis=-1, keepdims=True)

    e = jnp.exp(m2 - m1)
    denom = 1.0 + e
    c1 = 1.0 / denom
    c2 = e / denom

    idx_ref[...] = jnp.concatenate([i1, i2], axis=1)
    cw_ref[...] = jnp.concatenate([c1, c2], axis=1)

    z = logits - m1
    ez = jnp.exp(z)
    s = jnp.sum(ez, axis=-1, keepdims=True)
    p = ez / s
    psum = jnp.sum(p, axis=0)[None, :]
    ent = jnp.sum(jnp.log(s) - jnp.sum(p * z, axis=-1, keepdims=True))

    @pl.when(i == 0)
    def _init():
        psum_ref[...] = psum
        ent_ref[...] = jnp.full((1, 1), ent, jnp.float32)

    @pl.when(i != 0)
    def _acc():
        psum_ref[...] += psum
        ent_ref[...] += jnp.full((1, 1), ent, jnp.float32)


@jax.jit
def kernel(x, W):
    n_tok = x.shape[0]
    wt = W.T
    grid = (n_tok // BLK,)
    idx, cw, psum, ent = pl.pallas_call(
        _gate_kernel,
        grid=grid,
        in_specs=[
            pl.BlockSpec((BLK, D_MODEL), lambda i: (i, 0)),
            pl.BlockSpec((D_MODEL, N_EXP), lambda i: (0, 0)),
        ],
        out_specs=[
            pl.BlockSpec((BLK, 2), lambda i: (i, 0)),
            pl.BlockSpec((BLK, 2), lambda i: (i, 0)),
            pl.BlockSpec((1, N_EXP), lambda i: (0, 0)),
            pl.BlockSpec((1, 1), lambda i: (0, 0)),
        ],
        out_shape=[
            jax.ShapeDtypeStruct((n_tok, 2), jnp.int32),
            jax.ShapeDtypeStruct((n_tok, 2), jnp.float32),
            jax.ShapeDtypeStruct((1, N_EXP), jnp.float32),
            jax.ShapeDtypeStruct((1, 1), jnp.float32),
        ],
        compiler_params=pltpu.CompilerParams(
            dimension_semantics=("arbitrary",),
        ),
    )(x, wt)
    inv_n = jnp.float32(1.0 / n_tok)
    return idx, cw, psum[0] * inv_n, ent[0, 0] * inv_n


# speedup vs baseline: 1.7716x; 1.0349x over previous
import functools

import jax
import jax.numpy as jnp
from jax.experimental import pallas as pl
from jax.experimental.pallas import tpu as pltpu

D_MODEL = 2048
N_EXP = 64
N_TOK = 16384
BLK = 2048


def _gate_kernel(x_ref, w_ref, idx_ref, cw_ref, psum_ref, ent_ref):
    i = pl.program_id(0)
    xb = x_ref[...]
    wt = w_ref[...]
    logits = jnp.dot(xb, wt, preferred_element_type=jnp.float32)

    iota = jax.lax.broadcasted_iota(jnp.int32, logits.shape, 1)
    big = jnp.int32(N_EXP)

    m1 = jnp.max(logits, axis=-1, keepdims=True)
    i1 = jnp.min(jnp.where(logits == m1, iota, big), axis=-1, keepdims=True)
    l2 = jnp.where(iota == i1, -jnp.inf, logits)
    m2 = jnp.max(l2, axis=-1, keepdims=True)
    i2 = jnp.min(jnp.where(l2 == m2, iota, big), axis=-1, keepdims=True)

    e = jnp.exp(m2 - m1)
    denom = 1.0 + e
    c1 = 1.0 / denom
    c2 = e / denom

    idx_ref[...] = jnp.concatenate([i1, i2], axis=1)
    cw_ref[...] = jnp.concatenate([c1, c2], axis=1)

    z = logits - m1
    ez = jnp.exp(z)
    s = jnp.sum(ez, axis=-1, keepdims=True)
    p = ez / s
    psum = jnp.sum(p, axis=0)[None, :]
    ent = jnp.sum(jnp.log(s) - jnp.sum(p * z, axis=-1, keepdims=True))

    @pl.when(i == 0)
    def _init():
        psum_ref[...] = psum
        ent_ref[...] = jnp.full((1, 1), ent, jnp.float32)

    @pl.when(i != 0)
    def _acc():
        psum_ref[...] += psum
        ent_ref[...] += jnp.full((1, 1), ent, jnp.float32)


@jax.jit
def kernel(x, W):
    n_tok = x.shape[0]
    wt = W.T
    grid = (n_tok // BLK,)
    idx, cw, psum, ent = pl.pallas_call(
        _gate_kernel,
        grid=grid,
        in_specs=[
            pl.BlockSpec((BLK, D_MODEL), lambda i: (i, 0)),
            pl.BlockSpec((D_MODEL, N_EXP), lambda i: (0, 0)),
        ],
        out_specs=[
            pl.BlockSpec((BLK, 2), lambda i: (i, 0)),
            pl.BlockSpec((BLK, 2), lambda i: (i, 0)),
            pl.BlockSpec((1, N_EXP), lambda i: (0, 0)),
            pl.BlockSpec((1, 1), lambda i: (0, 0)),
        ],
        out_shape=[
            jax.ShapeDtypeStruct((n_tok, 2), jnp.int32),
            jax.ShapeDtypeStruct((n_tok, 2), jnp.float32),
            jax.ShapeDtypeStruct((1, N_EXP), jnp.float32),
            jax.ShapeDtypeStruct((1, 1), jnp.float32),
        ],
        compiler_params=pltpu.CompilerParams(
            dimension_semantics=("arbitrary",),
        ),
    )(x, wt)
    inv_n = jnp.float32(1.0 / n_tok)
    return idx, cw, psum[0] * inv_n, ent[0, 0] * inv_n
